# baseline (speedup 1.0000x reference)
import jax
import jax.numpy as jnp
from jax.experimental import pallas as pl
from jax.experimental.pallas import tpu as pltpu

B = 2
S = 256
H = 768
HEADS = 12
DH = H // HEADS
HH = H // 2
APAD = 128


def _gat_kernel(x_ref, t_hbm, wgat_hbm, asrc_ref, adst_ref, gb_ref, wproj_hbm,
                bp_ref, wfuse_hbm, bf_ref, out_ref,
                wgat_v, wproj_v, wfuse_v, t_v, out_v,
                sem_g1, sem_g2, sem_p, sem_f1, sem_f2, sem_t, sem_o):
    cp_g1 = pltpu.make_async_copy(wgat_hbm.at[pl.ds(0, HH)],
                                  wgat_v.at[pl.ds(0, HH)], sem_g1)
    cp_g2 = pltpu.make_async_copy(wgat_hbm.at[pl.ds(HH, HH)],
                                  wgat_v.at[pl.ds(HH, HH)], sem_g2)
    cp_p = pltpu.make_async_copy(wproj_hbm, wproj_v, sem_p)
    cp_t = pltpu.make_async_copy(t_hbm, t_v, sem_t)
    cp_f1 = pltpu.make_async_copy(wfuse_hbm.at[pl.ds(0, H)],
                                  wfuse_v.at[pl.ds(0, H)], sem_f1)
    cp_f2 = pltpu.make_async_copy(wfuse_hbm.at[pl.ds(H, H)],
                                  wfuse_v.at[pl.ds(H, H)], sem_f2)
    cp_g1.start()
    cp_g2.start()
    cp_p.start()
    cp_t.start()
    cp_f1.start()
    cp_f2.start()

    krow = jax.lax.broadcasted_iota(jnp.int32, (H, APAD), 0) // DH
    ccol = jax.lax.broadcasted_iota(jnp.int32, (H, APAD), 1)
    mask = (krow == ccol).astype(jnp.float32)

    cp_g1.wait()
    h_lo = [jnp.dot(x_ref[b][:, :HH], wgat_v[:HH, :],
                    preferred_element_type=jnp.float32) for b in range(B)]
    cp_g2.wait()
    hs = [h_lo[b] + jnp.dot(x_ref[b][:, HH:], wgat_v[HH:, :],
                            preferred_element_type=jnp.float32)
          for b in range(B)]

    attns = []
    for b in range(B):
        h = hs[b]
        q_src = h * asrc_ref[:]
        q_dst = h * adst_ref[:]
        a_srcT = jax.lax.dot_general(
            mask, q_src, (((0,), (1,)), ((), ())),
            preferred_element_type=jnp.float32)
        a_dst = jnp.dot(q_dst, mask,
                        preferred_element_type=jnp.float32)

        outs = []
        for hd in range(HEADS):
            row = a_srcT[hd:hd + 1, :]
            col = a_dst[:, hd:hd + 1]
            e = row + col
            e = jnp.where(e >= 0, e, 0.2 * e)
            p = jnp.exp(e)
            denom = jnp.sum(p, axis=1, keepdims=True)
            h_head = h[:, hd * DH:(hd + 1) * DH]
            acc = jnp.dot(p, h_head, preferred_element_type=jnp.float32)
            outs.append(acc / denom)
        attns.append(jnp.concatenate(outs, axis=1) + gb_ref[:])

    cp_p.wait()
    projs = [jnp.dot(a, wproj_v[:], preferred_element_type=jnp.float32)
             + bp_ref[:] for a in attns]
    cp_t.wait()
    cp_f1.wait()
    tops = [jnp.dot(t_v[b], wfuse_v[:H, :], preferred_element_type=jnp.float32)
            for b in range(B)]
    cp_f2.wait()
    cp_o = [pltpu.make_async_copy(out_v.at[b], out_ref.at[b], sem_o)
            for b in range(B)]
    for b in range(B):
        out_v[b] = (tops[b]
                    + jnp.dot(projs[b], wfuse_v[H:, :],
                              preferred_element_type=jnp.float32)
                    + bf_ref[:])
        cp_o[b].start()
    for b in range(B):
        cp_o[b].wait()


def kernel(hidden_states, transformer_output, W_gat, att_src, att_dst,
           gat_bias, W_proj, b_proj, W_fuse, b_fuse):
    asrc = att_src.reshape(1, H)
    adst = att_dst.reshape(1, H)
    gb = gat_bias.reshape(1, H)
    bp = b_proj.reshape(1, H)
    bf = b_fuse.reshape(1, H)

    vmem = pl.BlockSpec(memory_space=pltpu.MemorySpace.VMEM)
    hbm = pl.BlockSpec(memory_space=pltpu.MemorySpace.HBM)
    out = pl.pallas_call(
        _gat_kernel,
        in_specs=[
            vmem,
            hbm,
            hbm,
            vmem,
            vmem,
            vmem,
            hbm,
            vmem,
            hbm,
            vmem,
        ],
        out_specs=hbm,
        out_shape=jax.ShapeDtypeStruct((B, S, H), jnp.float32),
        scratch_shapes=[
            pltpu.VMEM((H, H), jnp.float32),
            pltpu.VMEM((H, H), jnp.float32),
            pltpu.VMEM((2 * H, H), jnp.float32),
            pltpu.VMEM((B, S, H), jnp.float32),
            pltpu.VMEM((B, S, H), jnp.float32),
            pltpu.SemaphoreType.DMA,
            pltpu.SemaphoreType.DMA,
            pltpu.SemaphoreType.DMA,
            pltpu.SemaphoreType.DMA,
            pltpu.SemaphoreType.DMA,
            pltpu.SemaphoreType.DMA,
            pltpu.SemaphoreType.DMA,
        ],
    )(hidden_states, transformer_output, W_gat, asrc, adst, gb, W_proj,
      bp, W_fuse, bf)
    return out
